# Initial kernel scaffold; baseline (speedup 1.0000x reference)
import jax
import jax.numpy as jnp
from jax import lax
from jax.experimental import pallas as pl
from jax.experimental.pallas import tpu as pltpu
from jax.experimental.pallas import tpu_sc as plsc

_B, _D, _T = 8, 512, 1024
_NCB, _K, _CD = 9, 8192, 256

_TB = 128
_NT = _T // _TB
_R = _B * _TB
_KC = 1024
_NKC = _K // _KC

_TB2 = 128
_NT2 = _T // _TB2
_R2 = _B * _TB2

_GW = 128
_NIDX = _B * _T

_INTERPRET = False


def _prep_body(cbt_ref, cbn_ref, csq_ref):
    cbt = cbt_ref[0]
    nrm = jnp.sqrt(jnp.sum(cbt * cbt, axis=0, keepdims=True))
    cbn = cbt / jnp.clip(nrm, 1e-12, None)
    cbn_ref[0] = cbn
    csq_ref[0] = jnp.sum(cbn * cbn, axis=0, keepdims=True)


def _prep(cbt):
    return pl.pallas_call(
        _prep_body,
        grid=(_NCB,),
        in_specs=[pl.BlockSpec((1, _CD, _K), lambda i: (i, 0, 0))],
        out_specs=[
            pl.BlockSpec((1, _CD, _K), lambda i: (i, 0, 0)),
            pl.BlockSpec((1, 1, _K), lambda i: (i, 0, 0)),
        ],
        out_shape=[
            jax.ShapeDtypeStruct((_NCB, _CD, _K), jnp.float32),
            jax.ShapeDtypeStruct((_NCB, 1, _K), jnp.float32),
        ],
        interpret=_INTERPRET,
    )(cbt)


def _encode_body(res_ref, iv_ref, ig_ref, ib_ref, cbn_ref, csq_ref,
                 ze_ref, lat_ref, idx_ref):
    v = iv_ref[0]
    g = ig_ref[0]
    w = g * v / jnp.sqrt(jnp.sum(v * v, axis=1, keepdims=True))
    res = res_ref[...].reshape(_R, _D)
    ze = lax.dot_general(res, w, (((1,), (1,)), ((), ())),
                         preferred_element_type=jnp.float32)
    ze = ze + ib_ref[0]
    ze3 = ze.reshape(_B, _TB, _CD)
    ze_ref[...] = ze3
    for b in range(_B):
        lat_ref[b] = ze3[b].T

    nrm = jnp.sqrt(jnp.sum(ze * ze, axis=1, keepdims=True))
    encn = ze / jnp.clip(nrm, 1e-12, None)
    a = jnp.sum(encn * encn, axis=1, keepdims=True)

    best_v = jnp.full((_R, 1), -jnp.inf, dtype=jnp.float32)
    best_i = jnp.zeros((_R, 1), jnp.int32)
    for kc in range(_NKC):
        cbn_c = cbn_ref[0, :, kc * _KC:(kc + 1) * _KC]
        csq_c = csq_ref[0, :, kc * _KC:(kc + 1) * _KC]
        m = lax.dot_general(encn, cbn_c, (((1,), (0,)), ((), ())),
                            preferred_element_type=jnp.float32)
        negd = -((a - 2.0 * m) + csq_c)
        mv = jnp.max(negd, axis=1, keepdims=True)
        ii = lax.broadcasted_iota(jnp.int32, (_R, _KC), 1)
        mi = jnp.min(jnp.where(negd == mv, ii, jnp.int32(2 ** 30)),
                     axis=1, keepdims=True) + kc * _KC
        upd = mv > best_v
        best_v = jnp.where(upd, mv, best_v)
        best_i = jnp.where(upd, mi, best_i)
    idx_ref[...] = best_i.reshape(_B, _TB)


def _encode(i, res, in_v3, ig3, ib3, cbn_all, csq_all):
    return pl.pallas_call(
        _encode_body,
        grid=(_NT,),
        in_specs=[
            pl.BlockSpec((_B, _TB, _D), lambda t: (0, t, 0)),
            pl.BlockSpec((1, _CD, _D), lambda t, i=i: (i, 0, 0)),
            pl.BlockSpec((1, _CD, 1), lambda t, i=i: (i, 0, 0)),
            pl.BlockSpec((1, 1, _CD), lambda t, i=i: (i, 0, 0)),
            pl.BlockSpec((1, _CD, _K), lambda t, i=i: (i, 0, 0)),
            pl.BlockSpec((1, 1, _K), lambda t, i=i: (i, 0, 0)),
        ],
        out_specs=[
            pl.BlockSpec((_B, _TB, _CD), lambda t: (0, t, 0)),
            pl.BlockSpec((_B, _CD, _TB), lambda t: (0, 0, t)),
            pl.BlockSpec((_B, _TB), lambda t: (0, t)),
        ],
        out_shape=[
            jax.ShapeDtypeStruct((_B, _T, _CD), jnp.float32),
            jax.ShapeDtypeStruct((_B, _CD, _T), jnp.float32),
            jax.ShapeDtypeStruct((_B, _T), jnp.int32),
        ],
        interpret=_INTERPRET,
    )(res, in_v3, ig3, ib3, cbn_all, csq_all)


def _gather(table, idx2):
    mesh = plsc.VectorSubcoreMesh(core_axis_name="c", subcore_axis_name="s")
    nw = mesh.num_cores * mesh.num_subcores
    nrows = idx2.shape[0]
    rpt = nrows // nw
    assert rpt * nw == nrows
    bpw = rpt * _GW
    nc = mesh.num_cores

    def body(table_hbm, idx_hbm, out_hbm, idx_v, rows_v, sem):
        wid = lax.axis_index("s") * nc + lax.axis_index("c")
        pltpu.sync_copy(idx_hbm.at[pl.ds(wid * rpt, rpt)], idx_v)
        cps = [
            pltpu.async_copy(table_hbm.at[idx_v.at[j]],
                             rows_v.at[pl.ds(j * _GW, _GW)], sem)
            for j in range(rpt)
        ]
        for c in cps:
            c.wait()
        pltpu.sync_copy(rows_v, out_hbm.at[pl.ds(wid * bpw, bpw)])

    k = pl.kernel(
        body,
        out_type=jax.ShapeDtypeStruct((_NIDX, _CD), jnp.float32),
        mesh=mesh,
        scratch_types=[
            pltpu.VMEM((rpt, _GW), jnp.int32),
            pltpu.VMEM((bpw, _CD), jnp.float32),
            pltpu.SemaphoreType.DMA,
        ],
    )
    return k(table, idx2)


def _finish_body(res_ref, zq_ref, ze_ref, zqi_ref, ov_ref, og_ref, ob_ref,
                 res_o, zq_o, loss_o):
    v = ov_ref[0]
    g = og_ref[0]
    w = g * v / jnp.sqrt(jnp.sum(v * v, axis=1, keepdims=True))
    zqi = zqi_ref[...].reshape(_R2, _CD)
    zqo = lax.dot_general(zqi, w, (((1,), (1,)), ((), ())),
                          preferred_element_type=jnp.float32)
    zqo = (zqo + ob_ref[0]).reshape(_B, _TB2, _D)
    res_o[...] = res_ref[...] - zqo
    zq_o[...] = zq_ref[...] + zqo
    d = ze_ref[...].reshape(_R2, _CD) - zqi
    s = jnp.sum(d * d)

    @pl.when(pl.program_id(0) == 0)
    def _():
        loss_o[0, 0] = s

    @pl.when(pl.program_id(0) != 0)
    def _():
        loss_o[0, 0] += s


def _finish(i, res, zq, ze, zqi, out_v3, og3, ob3):
    return pl.pallas_call(
        _finish_body,
        grid=(_NT2,),
        in_specs=[
            pl.BlockSpec((_B, _TB2, _D), lambda t: (0, t, 0)),
            pl.BlockSpec((_B, _TB2, _D), lambda t: (0, t, 0)),
            pl.BlockSpec((_B, _TB2, _CD), lambda t: (0, t, 0)),
            pl.BlockSpec((_B, _TB2, _CD), lambda t: (0, t, 0)),
            pl.BlockSpec((1, _D, _CD), lambda t, i=i: (i, 0, 0)),
            pl.BlockSpec((1, _D, 1), lambda t, i=i: (i, 0, 0)),
            pl.BlockSpec((1, 1, _D), lambda t, i=i: (i, 0, 0)),
        ],
        out_specs=[
            pl.BlockSpec((_B, _TB2, _D), lambda t: (0, t, 0)),
            pl.BlockSpec((_B, _TB2, _D), lambda t: (0, t, 0)),
            pl.BlockSpec((1, 1), lambda t: (0, 0)),
        ],
        out_shape=[
            jax.ShapeDtypeStruct((_B, _T, _D), jnp.float32),
            jax.ShapeDtypeStruct((_B, _T, _D), jnp.float32),
            jax.ShapeDtypeStruct((1, 1), jnp.float32),
        ],
        interpret=_INTERPRET,
    )(res, zq, ze, zqi, out_v3, og3, ob3)


def kernel(z, in_v, in_g, in_b, out_v, out_g, out_b, codebook):
    res = jnp.transpose(z, (0, 2, 1))
    zq = jnp.zeros_like(res)
    cbt = jnp.transpose(codebook, (0, 2, 1))
    cbn_all, csq_all = _prep(cbt)
    ig3 = in_g[..., None]
    ib3 = in_b[:, None, :]
    og3 = out_g[..., None]
    ob3 = out_b[:, None, :]

    codes, lats = [], []
    total = jnp.float32(0.0)
    for i in range(_NCB):
        ze, lat, idx = _encode(i, res, in_v, ig3, ib3, cbn_all, csq_all)
        idx2 = idx.reshape(_NIDX // _GW, _GW)
        zqi = _gather(codebook[i], idx2).reshape(_B, _T, _CD)
        res, zq, loss = _finish(i, res, zq, ze, zqi, out_v, og3, ob3)
        codes.append(idx)
        lats.append(lat)
        total = total + loss[0, 0] / jnp.float32(_CD * _T) / _B

    z_q = jnp.transpose(zq, (0, 2, 1))
    codes_o = jnp.stack(codes, axis=1)
    latents = jnp.concatenate(lats, axis=1)
    return (z_q, codes_o, latents, total, total)


# speedup vs baseline: 1.1622x; 1.1622x over previous
import jax
import jax.numpy as jnp
from jax import lax
from jax.experimental import pallas as pl
from jax.experimental.pallas import tpu as pltpu
from jax.experimental.pallas import tpu_sc as plsc

_B, _D, _T = 8, 512, 1024
_NCB, _K, _CD = 9, 8192, 256

_TB = 128
_NT = _T // _TB
_R = _B * _TB
_KC = 2048
_NKC = _K // _KC

_TB2 = 128
_NT2 = _T // _TB2
_R2 = _B * _TB2

_GW = 128
_NIDX = _B * _T

def _round_bf16(x):
    u = lax.bitcast_convert_type(x, jnp.uint32)
    lsb = jnp.bitwise_and(lax.shift_right_logical(u, jnp.uint32(16)),
                          jnp.uint32(1))
    r = jnp.bitwise_and(u + jnp.uint32(0x7FFF) + lsb, jnp.uint32(0xFFFF0000))
    return lax.bitcast_convert_type(r, jnp.float32)


def _prep_body(cbt_ref, cbn_ref, csq_ref):
    cbt = cbt_ref[0]
    nrm = jnp.sqrt(jnp.sum(cbt * cbt, axis=0, keepdims=True))
    cbn = cbt / jnp.clip(nrm, 1e-12, None)
    cbn_ref[0] = cbn
    csq_ref[0] = jnp.sum(cbn * cbn, axis=0, keepdims=True)


def _prep(cbt):
    return pl.pallas_call(
        _prep_body,
        grid=(_NCB,),
        in_specs=[pl.BlockSpec((1, _CD, _K), lambda i: (i, 0, 0))],
        out_specs=[
            pl.BlockSpec((1, _CD, _K), lambda i: (i, 0, 0)),
            pl.BlockSpec((1, 1, _K), lambda i: (i, 0, 0)),
        ],
        out_shape=[
            jax.ShapeDtypeStruct((_NCB, _CD, _K), jnp.float32),
            jax.ShapeDtypeStruct((_NCB, 1, _K), jnp.float32),
        ],
    )(cbt)


def _encode_body(res_ref, iv_ref, ig_ref, ib_ref, cbn_ref, csq_ref,
                 ze_ref, lat_ref, idx_ref):
    v = iv_ref[0]
    g = ig_ref[0]
    w = g * v / jnp.sqrt(jnp.sum(v * v, axis=1, keepdims=True))
    res = res_ref[...].reshape(_R, _D)
    ze = lax.dot_general(res, w, (((1,), (1,)), ((), ())),
                         preferred_element_type=jnp.float32)
    ze = ze + ib_ref[0]
    ze3 = ze.reshape(_B, _TB, _CD)
    ze_ref[...] = ze3
    for b in range(_B):
        lat_ref[b] = ze3[b].T

    nrm = jnp.sqrt(jnp.sum(ze * ze, axis=1, keepdims=True))
    encn = ze / jnp.clip(nrm, 1e-12, None)
    a = jnp.sum(encn * encn, axis=1, keepdims=True)
    enc2 = _round_bf16(2.0 * encn)

    best_v = jnp.full((_R, 1), -jnp.inf, dtype=jnp.float32)
    best_i = jnp.zeros((_R, 1), jnp.int32)
    for kc in range(_NKC):
        cbn_c = cbn_ref[0, :, kc * _KC:(kc + 1) * _KC]
        csq_c = csq_ref[0, :, kc * _KC:(kc + 1) * _KC]
        m2 = lax.dot_general(enc2, cbn_c, (((1,), (0,)), ((), ())),
                             preferred_element_type=jnp.float32)
        negd = -((a - m2) + csq_c)
        mv = jnp.max(negd, axis=1, keepdims=True)
        ii = lax.broadcasted_iota(jnp.int32, (_R, _KC), 1)
        mi = jnp.min(jnp.where(negd == mv, ii, jnp.int32(2 ** 30)),
                     axis=1, keepdims=True) + kc * _KC
        upd = mv > best_v
        best_v = jnp.where(upd, _round_bf16(mv), best_v)
        best_i = jnp.where(upd, mi, best_i)
    idx_ref[...] = best_i.reshape(_B, _TB)


def _encode(i, res, in_v3, ig3, ib3, cbn_all, csq_all):
    return pl.pallas_call(
        _encode_body,
        grid=(_NT,),
        in_specs=[
            pl.BlockSpec((_B, _TB, _D), lambda t: (0, t, 0)),
            pl.BlockSpec((1, _CD, _D), lambda t, i=i: (i, 0, 0)),
            pl.BlockSpec((1, _CD, 1), lambda t, i=i: (i, 0, 0)),
            pl.BlockSpec((1, 1, _CD), lambda t, i=i: (i, 0, 0)),
            pl.BlockSpec((1, _CD, _K), lambda t, i=i: (i, 0, 0)),
            pl.BlockSpec((1, 1, _K), lambda t, i=i: (i, 0, 0)),
        ],
        out_specs=[
            pl.BlockSpec((_B, _TB, _CD), lambda t: (0, t, 0)),
            pl.BlockSpec((_B, _CD, _TB), lambda t: (0, 0, t)),
            pl.BlockSpec((_B, _TB), lambda t: (0, t)),
        ],
        out_shape=[
            jax.ShapeDtypeStruct((_B, _T, _CD), jnp.float32),
            jax.ShapeDtypeStruct((_B, _CD, _T), jnp.float32),
            jax.ShapeDtypeStruct((_B, _T), jnp.int32),
        ],
    )(res, in_v3, ig3, ib3, cbn_all, csq_all)


def _gather(table, idx2):
    mesh = plsc.VectorSubcoreMesh(core_axis_name="c", subcore_axis_name="s")
    nw = mesh.num_cores * mesh.num_subcores
    nrows = idx2.shape[0]
    rpt = nrows // nw
    assert rpt * nw == nrows
    bpw = rpt * _GW
    nc = mesh.num_cores

    def body(table_hbm, idx_hbm, out_hbm, idx_v, rows_v, sem):
        wid = lax.axis_index("s") * nc + lax.axis_index("c")
        pltpu.sync_copy(idx_hbm.at[pl.ds(wid * rpt, rpt)], idx_v)
        cps = [
            pltpu.async_copy(table_hbm.at[idx_v.at[j]],
                             rows_v.at[pl.ds(j * _GW, _GW)], sem)
            for j in range(rpt)
        ]
        for c in cps:
            c.wait()
        pltpu.sync_copy(rows_v, out_hbm.at[pl.ds(wid * bpw, bpw)])

    k = pl.kernel(
        body,
        out_type=jax.ShapeDtypeStruct((_NIDX, _CD), jnp.float32),
        mesh=mesh,
        scratch_types=[
            pltpu.VMEM((rpt, _GW), jnp.int32),
            pltpu.VMEM((bpw, _CD), jnp.float32),
            pltpu.SemaphoreType.DMA,
        ],
    )
    return k(table, idx2)


def _finish_body(res_ref, zq_ref, ze_ref, zqi_ref, ov_ref, og_ref, ob_ref,
                 res_o, zq_o, loss_o):
    v = ov_ref[0]
    g = og_ref[0]
    w = g * v / jnp.sqrt(jnp.sum(v * v, axis=1, keepdims=True))
    zqi = zqi_ref[...].reshape(_R2, _CD)
    zqo = lax.dot_general(zqi, w, (((1,), (1,)), ((), ())),
                          preferred_element_type=jnp.float32)
    zqo = (zqo + ob_ref[0]).reshape(_B, _TB2, _D)
    res_o[...] = res_ref[...] - zqo
    zq_o[...] = zq_ref[...] + zqo
    d = ze_ref[...].reshape(_R2, _CD) - zqi
    s = jnp.sum(d * d)

    @pl.when(pl.program_id(0) == 0)
    def _():
        loss_o[...] = s.reshape(1, 1)

    @pl.when(pl.program_id(0) != 0)
    def _():
        loss_o[...] = loss_o[...] + s.reshape(1, 1)


def _finish(i, res, zq, ze, zqi, out_v3, og3, ob3):
    return pl.pallas_call(
        _finish_body,
        grid=(_NT2,),
        in_specs=[
            pl.BlockSpec((_B, _TB2, _D), lambda t: (0, t, 0)),
            pl.BlockSpec((_B, _TB2, _D), lambda t: (0, t, 0)),
            pl.BlockSpec((_B, _TB2, _CD), lambda t: (0, t, 0)),
            pl.BlockSpec((_B, _TB2, _CD), lambda t: (0, t, 0)),
            pl.BlockSpec((1, _D, _CD), lambda t, i=i: (i, 0, 0)),
            pl.BlockSpec((1, _D, 1), lambda t, i=i: (i, 0, 0)),
            pl.BlockSpec((1, 1, _D), lambda t, i=i: (i, 0, 0)),
        ],
        out_specs=[
            pl.BlockSpec((_B, _TB2, _D), lambda t: (0, t, 0)),
            pl.BlockSpec((_B, _TB2, _D), lambda t: (0, t, 0)),
            pl.BlockSpec((1, 1), lambda t: (0, 0)),
        ],
        out_shape=[
            jax.ShapeDtypeStruct((_B, _T, _D), jnp.float32),
            jax.ShapeDtypeStruct((_B, _T, _D), jnp.float32),
            jax.ShapeDtypeStruct((1, 1), jnp.float32),
        ],
    )(res, zq, ze, zqi, out_v3, og3, ob3)


def kernel(z, in_v, in_g, in_b, out_v, out_g, out_b, codebook):
    res = jnp.transpose(z, (0, 2, 1))
    zq = jnp.zeros_like(res)
    cbt = jnp.transpose(codebook, (0, 2, 1))
    cbn_all, csq_all = _prep(cbt)
    ig3 = in_g[..., None]
    ib3 = in_b[:, None, :]
    og3 = out_g[..., None]
    ob3 = out_b[:, None, :]

    codes, lats = [], []
    total = jnp.float32(0.0)
    for i in range(_NCB):
        ze, lat, idx = _encode(i, res, in_v, ig3, ib3, cbn_all, csq_all)
        idx2 = idx.reshape(_NIDX // _GW, _GW)
        zqi = _gather(codebook[i], idx2).reshape(_B, _T, _CD)
        res, zq, loss = _finish(i, res, zq, ze, zqi, out_v, og3, ob3)
        codes.append(idx)
        lats.append(lat)
        total = total + loss[0, 0] / jnp.float32(_CD * _T) / _B

    z_q = jnp.transpose(zq, (0, 2, 1))
    codes_o = jnp.stack(codes, axis=1)
    latents = jnp.concatenate(lats, axis=1)
    return (z_q, codes_o, latents, total, total)
